# baseline (speedup 1.0000x reference)
import functools
import math

import jax
import jax.numpy as jnp
from jax import lax
from jax.experimental import pallas as pl
from jax.experimental.pallas import tpu as pltpu
from jax.experimental.pallas import tpu_sc as plsc

N = 10000
E = 320000
D = 128
NPAD = 10240
RB = 2048
NBLK = NPAD // RB
NC = 2
NS = 16
NW = NC * NS


_mesh = plsc.VectorSubcoreMesh(core_axis_name="c", subcore_axis_name="s")

RCH = 500
NCHUNK = E // NW // RCH
RPT = NPAD // NS
ZB = 40


DH = 64
DG = 16


def _sc_rowpass_body(dw, table_hbm, src_hbm, dst_hbm, out_hbm,
                     src_v, dst_v, rows0_v, rows1_v, zb_v, acc_sp,
                     sem0, sem1):
    cid = lax.axis_index("c")
    sid = lax.axis_index("s")
    wid = cid * NS + sid

    def _zero_zb(i, _):
        for l in range(dw // 16):
            zb_v[i, pl.ds(l * 16, 16)] = jnp.zeros((16,), jnp.float32)
        return 0
    lax.fori_loop(0, ZB, _zero_zb, 0)

    row0 = sid * RPT
    for j in range(RPT // ZB):
        pltpu.sync_copy(zb_v, acc_sp.at[pl.ds(row0 + j * ZB, ZB)])

    pltpu.sync_copy(src_hbm.at[wid], src_v)
    pltpu.sync_copy(dst_hbm.at[wid], dst_v)
    plsc.subcore_barrier()

    bufs = (rows0_v, rows1_v)
    sems = (sem0, sem1)
    g = pltpu.async_copy(table_hbm.at[src_v.at[0]], bufs[0], sems[0])
    for j in range(NCHUNK):
        g.wait()
        if j + 1 < NCHUNK:
            g = pltpu.async_copy(table_hbm.at[src_v.at[j + 1]],
                                 bufs[(j + 1) % 2], sems[(j + 1) % 2])
        pltpu.sync_copy(bufs[j % 2], acc_sp.at[dst_v.at[j]], add=True)
    plsc.subcore_barrier()

    for j in range(RPT // ZB):
        pltpu.sync_copy(acc_sp.at[pl.ds(row0 + j * ZB, ZB)], zb_v)
        pltpu.sync_copy(zb_v, out_hbm.at[cid, pl.ds(row0 + j * ZB, ZB)])


@functools.cache
def _make_rowpass(dw):
    @functools.partial(
        pl.kernel, mesh=_mesh,
        out_type=jax.ShapeDtypeStruct((NC, NPAD, dw), jnp.float32),
        compiler_params=pltpu.CompilerParams(use_tc_tiling_on_sc=False),
        scratch_types=[
            pltpu.VMEM((NCHUNK, RCH), jnp.int32),
            pltpu.VMEM((NCHUNK, RCH), jnp.int32),
            pltpu.VMEM((RCH, dw), jnp.float32),
            pltpu.VMEM((RCH, dw), jnp.float32),
            pltpu.VMEM((ZB, dw), jnp.float32),
            pltpu.VMEM_SHARED((NPAD, dw), jnp.float32),
            pltpu.SemaphoreType.DMA,
            pltpu.SemaphoreType.DMA,
        ],
    )
    def _k(table_hbm, src_hbm, dst_hbm, out_hbm, *rest):
        _sc_rowpass_body(dw, table_hbm, src_hbm, dst_hbm, out_hbm, *rest)

    return _k


def _sc_rowpass(table, src_t, dst_t):
    return _make_rowpass(table.shape[1])(table, src_t, dst_t)


def _sc_rowpass2_body(ta_hbm, tb_hbm, src_hbm, dst_hbm, outa_hbm, outb_hbm,
                      src_v, dst_v, rows0_v, rows1_v, zb_v, wb_v, acc_sp,
                      sem0, sem1):
    cid = lax.axis_index("c")
    sid = lax.axis_index("s")
    wid = cid * NS + sid
    row0 = sid * RPT

    def _zero_zb(i, _):
        for l in range(DH // 16):
            zb_v[i, pl.ds(l * 16, 16)] = jnp.zeros((16,), jnp.float32)
        return 0
    lax.fori_loop(0, ZB, _zero_zb, 0)

    def _zero_acc():
        for j in range(RPT // ZB):
            pltpu.sync_copy(zb_v, acc_sp.at[pl.ds(row0 + j * ZB, ZB)])

    def _writeback(out_hbm):
        for j in range(RPT // ZB):
            pltpu.sync_copy(acc_sp.at[pl.ds(row0 + j * ZB, ZB)], wb_v)
            pltpu.sync_copy(wb_v, out_hbm.at[cid, pl.ds(row0 + j * ZB, ZB)])

    _zero_acc()
    pltpu.sync_copy(src_hbm.at[wid], src_v)
    pltpu.sync_copy(dst_hbm.at[wid], dst_v)
    plsc.subcore_barrier()

    bufs = (rows0_v, rows1_v)
    sems = (sem0, sem1)
    tabs = (ta_hbm, tb_hbm)
    g = pltpu.async_copy(ta_hbm.at[src_v.at[0]], bufs[0], sems[0])
    for j in range(2 * NCHUNK):
        g.wait()
        if j + 1 < 2 * NCHUNK:
            nj = (j + 1) % NCHUNK
            g = pltpu.async_copy(tabs[(j + 1) // NCHUNK].at[src_v.at[nj]],
                                 bufs[(j + 1) % 2], sems[(j + 1) % 2])
        if j == NCHUNK:
            plsc.subcore_barrier()
            _writeback(outa_hbm)
            _zero_acc()
            plsc.subcore_barrier()
        pltpu.sync_copy(bufs[j % 2], acc_sp.at[dst_v.at[j % NCHUNK]],
                        add=True)
    plsc.subcore_barrier()
    _writeback(outb_hbm)


@functools.partial(
    pl.kernel, mesh=_mesh,
    out_type=[jax.ShapeDtypeStruct((NC, NPAD, DH), jnp.float32),
              jax.ShapeDtypeStruct((NC, NPAD, DH), jnp.float32)],
    compiler_params=pltpu.CompilerParams(use_tc_tiling_on_sc=False),
    scratch_types=[
        pltpu.VMEM((NCHUNK, RCH), jnp.int32),
        pltpu.VMEM((NCHUNK, RCH), jnp.int32),
        pltpu.VMEM((RCH, DH), jnp.float32),
        pltpu.VMEM((RCH, DH), jnp.float32),
        pltpu.VMEM((ZB, DH), jnp.float32),
        pltpu.VMEM((ZB, DH), jnp.float32),
        pltpu.VMEM_SHARED((NPAD, DH), jnp.float32),
        pltpu.SemaphoreType.DMA,
        pltpu.SemaphoreType.DMA,
    ],
)
def _sc_rowpass2(ta, tb, src_hbm, dst_hbm, outa, outb, *rest):
    _sc_rowpass2_body(ta, tb, src_hbm, dst_hbm, outa, outb, *rest)


def _safe_rsqrt(d):
    return jnp.where(d > 0, lax.rsqrt(jnp.where(d > 0, d, 1.0)), 0.0)


def _stage_a_body(x_ref, w_ref, wd0_ref, wd1_ref, keep_ref,
                  xs0_ref, xs1_ref, dc_ref, di_ref):
    wd = wd0_ref[...] + wd1_ref[...]
    keep = keep_ref[...]
    dc = _safe_rsqrt(keep * (wd + 1.0))
    di = _safe_rsqrt(keep * wd)
    dc_ref[...] = dc
    di_ref[...] = di
    xs = jnp.dot(x_ref[...], w_ref[...], preferred_element_type=jnp.float32)
    xs = xs * dc
    xs0_ref[...] = xs[:, :DH]
    xs1_ref[...] = xs[:, DH:]


def _stage_a(x, w, wd0, wd1, keep):
    return pl.pallas_call(
        _stage_a_body,
        grid=(NBLK,),
        in_specs=[
            pl.BlockSpec((RB, D), lambda i: (i, 0)),
            pl.BlockSpec((D, D), lambda i: (0, 0)),
            pl.BlockSpec((RB, 1), lambda i: (i, 0)),
            pl.BlockSpec((RB, 1), lambda i: (i, 0)),
            pl.BlockSpec((RB, 1), lambda i: (i, 0)),
        ],
        out_specs=[
            pl.BlockSpec((RB, DH), lambda i: (i, 0)),
            pl.BlockSpec((RB, DH), lambda i: (i, 0)),
            pl.BlockSpec((RB, 1), lambda i: (i, 0)),
            pl.BlockSpec((RB, 1), lambda i: (i, 0)),
        ],
        out_shape=[
            jax.ShapeDtypeStruct((NPAD, DH), jnp.float32),
            jax.ShapeDtypeStruct((NPAD, DH), jnp.float32),
            jax.ShapeDtypeStruct((NPAD, 1), jnp.float32),
            jax.ShapeDtypeStruct((NPAD, 1), jnp.float32),
        ],
    )(x, w, wd0, wd1, keep)


def _stage_b_body(a0_ref, a1_ref, xs0_ref, xs1_ref, dc_ref, di_ref, b_ref,
                  h_ref, hsb0_ref, hsb1_ref):
    acc = jnp.concatenate([a0_ref[0] + a0_ref[1], a1_ref[0] + a1_ref[1]],
                          axis=1)
    xs = jnp.concatenate([xs0_ref[...], xs1_ref[...]], axis=1)
    h = jax.nn.relu(dc_ref[...] * (acc + xs) + b_ref[0:1, :])
    h_ref[...] = h
    hsb = di_ref[...] * h
    hsb0_ref[...] = hsb[:, :DH]
    hsb1_ref[...] = hsb[:, DH:]


def _stage_b(a0, a1, xs0, xs1, dc, di, b8):
    return pl.pallas_call(
        _stage_b_body,
        grid=(NBLK,),
        in_specs=[
            pl.BlockSpec((NC, RB, DH), lambda i: (0, i, 0)),
            pl.BlockSpec((NC, RB, DH), lambda i: (0, i, 0)),
            pl.BlockSpec((RB, DH), lambda i: (i, 0)),
            pl.BlockSpec((RB, DH), lambda i: (i, 0)),
            pl.BlockSpec((RB, 1), lambda i: (i, 0)),
            pl.BlockSpec((RB, 1), lambda i: (i, 0)),
            pl.BlockSpec((8, D), lambda i: (0, 0)),
        ],
        out_specs=[
            pl.BlockSpec((RB, D), lambda i: (i, 0)),
            pl.BlockSpec((RB, DH), lambda i: (i, 0)),
            pl.BlockSpec((RB, DH), lambda i: (i, 0)),
        ],
        out_shape=[
            jax.ShapeDtypeStruct((NPAD, D), jnp.float32),
            jax.ShapeDtypeStruct((NPAD, DH), jnp.float32),
            jax.ShapeDtypeStruct((NPAD, DH), jnp.float32),
        ],
    )(a0, a1, xs0, xs1, dc, di, b8)


def _stage_c_body(b0_ref, b1_ref, h_ref, di_ref, keep_ref, s_ref):
    accb = jnp.concatenate([b0_ref[0] + b0_ref[1], b1_ref[0] + b1_ref[1]],
                           axis=1)
    agg = h_ref[...] - di_ref[...] * accb
    s = jnp.sum(jnp.abs(agg), axis=1, keepdims=True)
    s_ref[...] = jnp.where(keep_ref[...] > 0, s, -jnp.inf)


def _stage_c(b0, b1, h, di, keep):
    return pl.pallas_call(
        _stage_c_body,
        grid=(NBLK,),
        in_specs=[
            pl.BlockSpec((NC, RB, DH), lambda i: (0, i, 0)),
            pl.BlockSpec((NC, RB, DH), lambda i: (0, i, 0)),
            pl.BlockSpec((RB, D), lambda i: (i, 0)),
            pl.BlockSpec((RB, 1), lambda i: (i, 0)),
            pl.BlockSpec((RB, 1), lambda i: (i, 0)),
        ],
        out_specs=pl.BlockSpec((RB, 1), lambda i: (i, 0)),
        out_shape=jax.ShapeDtypeStruct((NPAD, 1), jnp.float32),
    )(b0, b1, h, di, keep)


def _topk_body(s_ref, keep_ref, *, k):
    sb = lax.bitcast_convert_type(s_ref[...], jnp.int32)
    ikey = sb ^ (lax.shift_right_arithmetic(sb, 31) & jnp.int32(0x7FFFFFFF))
    msb = jnp.int32(-2147483648)

    def bit_body(i, t):
        bit = lax.shift_left(jnp.int32(1), 31 - i)
        t2 = t | bit
        thr = t2 ^ msb
        cnt = jnp.sum((ikey >= thr).astype(jnp.int32))
        return jnp.where(cnt >= k, t2, t)

    t = lax.fori_loop(0, 32, bit_body, jnp.int32(0))
    thr = t ^ msb
    gt = ikey > thr
    tie = ikey == thr
    need = (k - jnp.sum(gt.astype(jnp.int32))).astype(jnp.float32)

    tie_f = tie.astype(jnp.float32)
    c_i = lax.broadcasted_iota(jnp.int32, (D, D), 0)
    c_j = lax.broadcasted_iota(jnp.int32, (D, D), 1)
    u_strict = (c_i < c_j).astype(jnp.float32)
    prefix = jnp.dot(tie_f, u_strict, preferred_element_type=jnp.float32)
    r_i = lax.broadcasted_iota(jnp.int32, (NPAD // D, NPAD // D), 0)
    r_j = lax.broadcasted_iota(jnp.int32, (NPAD // D, NPAD // D), 1)
    l_strict = (r_j < r_i).astype(jnp.float32)
    rowtot = jnp.sum(tie_f, axis=1, keepdims=True)
    offs = jnp.dot(l_strict, rowtot, preferred_element_type=jnp.float32)
    rank = offs + prefix
    keep = gt | (tie & (rank < need))
    keep_ref[...] = keep.astype(jnp.float32)


def _topk(score80, k):
    return pl.pallas_call(
        functools.partial(_topk_body, k=k),
        out_shape=jax.ShapeDtypeStruct((NPAD // D, D), jnp.float32),
    )(score80)


def _readout_body(h_ref, keep_ref, o_ref):
    @pl.when(pl.program_id(0) == 0)
    def _():
        o_ref[...] = jnp.zeros((8, D), jnp.float32)
        o_ref[0:1, :] = jnp.full((1, D), -jnp.inf, jnp.float32)

    m = keep_ref[...] > 0
    hm = jnp.where(m, h_ref[...], -jnp.inf)
    hs = jnp.where(m, h_ref[...], 0.0)
    o_ref[0:1, :] = jnp.maximum(o_ref[0:1, :], jnp.max(hm, axis=0, keepdims=True))
    o_ref[1:2, :] = o_ref[1:2, :] + jnp.sum(hs, axis=0, keepdims=True)


def _readout(h, keep):
    return pl.pallas_call(
        _readout_body,
        grid=(NBLK,),
        in_specs=[
            pl.BlockSpec((RB, D), lambda i: (i, 0)),
            pl.BlockSpec((RB, 1), lambda i: (i, 0)),
        ],
        out_specs=pl.BlockSpec((8, D), lambda i: (0, 0)),
        out_shape=jax.ShapeDtypeStruct((8, D), jnp.float32),
    )(h, keep)


def _mlp_body(r1_ref, r2_ref, r3_ref, w1_ref, b1_ref, w2_ref, b2_ref,
              w3_ref, b3_ref, o_ref, *, k1, k2):
    def ro(r_ref, k):
        mx = r_ref[0:1, :]
        mn = r_ref[1:2, :] * (1.0 / k)
        return jnp.concatenate([mx, mn], axis=1)

    x1 = jax.nn.relu(ro(r1_ref, k1))
    x2 = jax.nn.relu(ro(r2_ref, k2))
    x3 = jax.nn.relu(ro(r3_ref, k2))
    z = x1 + x2 + x3
    z = jax.nn.relu(jnp.dot(z, w1_ref[...], preferred_element_type=jnp.float32)
                    + b1_ref[0:1, :])
    z = jax.nn.relu(jnp.dot(z, w2_ref[...], preferred_element_type=jnp.float32)
                    + b2_ref[0:1, :])
    z = jnp.dot(z, w3_ref[...], preferred_element_type=jnp.float32) + b3_ref[0:1, :]
    col = lax.broadcasted_iota(jnp.int32, (1, D), 1)
    zm = jnp.where(col < 6, z, -jnp.inf)
    mx = jnp.max(zm, axis=1, keepdims=True)
    ze = jnp.where(col < 6, jnp.exp(zm - mx), 0.0)
    lse = jnp.log(jnp.sum(ze, axis=1, keepdims=True)) + mx
    o_ref[...] = zm - lse


def _mlp(r1, r2, r3, w1p, b1p, w2p, b2p, w3p, b3p, k1, k2):
    return pl.pallas_call(
        functools.partial(_mlp_body, k1=float(k1), k2=float(k2)),
        out_shape=jax.ShapeDtypeStruct((1, D), jnp.float32),
    )(r1, r2, r3, w1p, b1p, w2p, b2p, w3p, b3p)


def _pad_rows(a, rows):
    return jnp.pad(a, ((0, rows - a.shape[0]), (0, 0)))


def kernel(x, edge_index, batch, W1, b1, W2, b2, W3, b3,
           lin1_W, lin1_b, lin2_W, lin2_b, lin3_W, lin3_b):
    src = edge_index[0].astype(jnp.int32)
    dst = edge_index[1].astype(jnp.int32)
    src_t = src.reshape(NW, NCHUNK, RCH)
    dst_t = dst.reshape(NW, NCHUNK, RCH)

    xp = _pad_rows(x, NPAD)
    ones = jnp.ones((NPAD, 1), jnp.float32)

    k1 = int(math.ceil(0.5 * N))
    k2 = int(math.ceil(0.5 * k1))

    def b_pad8(b):
        return jnp.broadcast_to(b[None, :], (8, b.shape[0]))

    def layer(h_in, W, b, keep_col, want_score):
        keep16 = jnp.broadcast_to(keep_col, (NPAD, DG))
        wd = _sc_rowpass(keep16, dst_t, src_t)
        xs0, xs1, dc, di = _stage_a(h_in, W, wd[0, :, 0:1], wd[1, :, 0:1],
                                    keep_col)
        a0, a1 = _sc_rowpass2(xs0, xs1, src_t, dst_t)
        h, hsb0, hsb1 = _stage_b(a0, a1, xs0, xs1, dc, di, b_pad8(b))
        if not want_score:
            return h, None
        b0, b1 = _sc_rowpass2(hsb0, hsb1, src_t, dst_t)
        score = _stage_c(b0, b1, h, di, keep_col)
        return h, score

    h1, s1 = layer(xp, W1, b1, ones, True)
    keep1 = _topk(s1.reshape(NPAD // D, D), k1).reshape(NPAD, 1)
    h2, s2 = layer(h1, W2, b2, keep1, True)
    keep2 = _topk(s2.reshape(NPAD // D, D), k2).reshape(NPAD, 1)
    h3, _ = layer(h2, W3, b3, keep2, False)

    r1 = _readout(h1, keep1)
    r2 = _readout(h2, keep2)
    r3 = _readout(h3, keep2)

    w3p = jnp.pad(lin3_W, ((0, 0), (0, D - lin3_W.shape[1])))
    b3p = jnp.pad(lin3_b, (0, D - lin3_b.shape[0]))
    w2p = lin2_W
    out = _mlp(r1, r2, r3,
               lin1_W, b_pad8(lin1_b),
               w2p, b_pad8(jnp.pad(lin2_b, (0, 0))),
               w3p, b_pad8(b3p), k1, k2)
    return out[:, :6]


# speedup vs baseline: 1.0183x; 1.0183x over previous
import functools
import math

import jax
import jax.numpy as jnp
from jax import lax
from jax.experimental import pallas as pl
from jax.experimental.pallas import tpu as pltpu
from jax.experimental.pallas import tpu_sc as plsc

N = 10000
E = 320000
D = 128
NPAD = 10240
RB = 2048
NBLK = NPAD // RB
NC = 2
NS = 16
NW = NC * NS


_mesh = plsc.VectorSubcoreMesh(core_axis_name="c", subcore_axis_name="s")

RCH = 500
NCHUNK = E // NW // RCH
RPT = NPAD // NS
ZB = 40


DH = 64
DG = 16


def _sc_rowpass_body(dw, table_hbm, src_hbm, dst_hbm, out_hbm,
                     src_v, dst_v, rows0_v, rows1_v, zb_v, acc_sp,
                     sem0, sem1):
    cid = lax.axis_index("c")
    sid = lax.axis_index("s")
    wid = cid * NS + sid

    def _zero_zb(i, _):
        for l in range(dw // 16):
            zb_v[i, pl.ds(l * 16, 16)] = jnp.zeros((16,), jnp.float32)
        return 0
    lax.fori_loop(0, ZB, _zero_zb, 0)

    row0 = sid * RPT
    for j in range(RPT // ZB):
        pltpu.sync_copy(zb_v, acc_sp.at[pl.ds(row0 + j * ZB, ZB)])

    pltpu.sync_copy(src_hbm.at[wid], src_v)
    pltpu.sync_copy(dst_hbm.at[wid], dst_v)
    plsc.subcore_barrier()

    bufs = (rows0_v, rows1_v)
    sems = (sem0, sem1)
    g = pltpu.async_copy(table_hbm.at[src_v.at[0]], bufs[0], sems[0])
    for j in range(NCHUNK):
        g.wait()
        if j + 1 < NCHUNK:
            g = pltpu.async_copy(table_hbm.at[src_v.at[j + 1]],
                                 bufs[(j + 1) % 2], sems[(j + 1) % 2])
        pltpu.sync_copy(bufs[j % 2], acc_sp.at[dst_v.at[j]], add=True)
    plsc.subcore_barrier()

    for j in range(RPT // ZB):
        pltpu.sync_copy(acc_sp.at[pl.ds(row0 + j * ZB, ZB)], zb_v)
        pltpu.sync_copy(zb_v, out_hbm.at[cid, pl.ds(row0 + j * ZB, ZB)])


@functools.cache
def _make_rowpass(dw):
    @functools.partial(
        pl.kernel, mesh=_mesh,
        out_type=jax.ShapeDtypeStruct((NC, NPAD, dw), jnp.float32),
        compiler_params=pltpu.CompilerParams(use_tc_tiling_on_sc=False),
        scratch_types=[
            pltpu.VMEM((NCHUNK, RCH), jnp.int32),
            pltpu.VMEM((NCHUNK, RCH), jnp.int32),
            pltpu.VMEM((RCH, dw), jnp.float32),
            pltpu.VMEM((RCH, dw), jnp.float32),
            pltpu.VMEM((ZB, dw), jnp.float32),
            pltpu.VMEM_SHARED((NPAD, dw), jnp.float32),
            pltpu.SemaphoreType.DMA,
            pltpu.SemaphoreType.DMA,
        ],
    )
    def _k(table_hbm, src_hbm, dst_hbm, out_hbm, *rest):
        _sc_rowpass_body(dw, table_hbm, src_hbm, dst_hbm, out_hbm, *rest)

    return _k


def _sc_rowpass(table, src_t, dst_t):
    return _make_rowpass(table.shape[1])(table, src_t, dst_t)


def _sc_rowpass2_body(ta_hbm, tb_hbm, src_hbm, dst_hbm, outa_hbm, outb_hbm,
                      src_v, dst_v, rows0_v, rows1_v, zb_v, wb_v, acc_sp,
                      sem0, sem1):
    cid = lax.axis_index("c")
    sid = lax.axis_index("s")
    wid = cid * NS + sid
    row0 = sid * RPT

    def _zero_zb(i, _):
        for l in range(DH // 16):
            zb_v[i, pl.ds(l * 16, 16)] = jnp.zeros((16,), jnp.float32)
        return 0
    lax.fori_loop(0, ZB, _zero_zb, 0)

    def _zero_acc():
        for j in range(RPT // ZB):
            pltpu.sync_copy(zb_v, acc_sp.at[pl.ds(row0 + j * ZB, ZB)])

    def _writeback(out_hbm):
        pltpu.sync_copy(acc_sp.at[pl.ds(row0, RPT)],
                        out_hbm.at[cid, pl.ds(row0, RPT)])

    _zero_acc()
    pltpu.sync_copy(src_hbm.at[wid], src_v)
    pltpu.sync_copy(dst_hbm.at[wid], dst_v)
    plsc.subcore_barrier()

    bufs = (rows0_v, rows1_v)
    sems = (sem0, sem1)
    tabs = (ta_hbm, tb_hbm)
    g = pltpu.async_copy(ta_hbm.at[src_v.at[0]], bufs[0], sems[0])
    for j in range(2 * NCHUNK):
        g.wait()
        if j + 1 < 2 * NCHUNK:
            nj = (j + 1) % NCHUNK
            g = pltpu.async_copy(tabs[(j + 1) // NCHUNK].at[src_v.at[nj]],
                                 bufs[(j + 1) % 2], sems[(j + 1) % 2])
        if j == NCHUNK:
            plsc.subcore_barrier()
            _writeback(outa_hbm)
            _zero_acc()
            plsc.subcore_barrier()
        pltpu.sync_copy(bufs[j % 2], acc_sp.at[dst_v.at[j % NCHUNK]],
                        add=True)
    plsc.subcore_barrier()
    _writeback(outb_hbm)


@functools.partial(
    pl.kernel, mesh=_mesh,
    out_type=[jax.ShapeDtypeStruct((NC, NPAD, DH), jnp.float32),
              jax.ShapeDtypeStruct((NC, NPAD, DH), jnp.float32)],
    compiler_params=pltpu.CompilerParams(use_tc_tiling_on_sc=False),
    scratch_types=[
        pltpu.VMEM((NCHUNK, RCH), jnp.int32),
        pltpu.VMEM((NCHUNK, RCH), jnp.int32),
        pltpu.VMEM((RCH, DH), jnp.float32),
        pltpu.VMEM((RCH, DH), jnp.float32),
        pltpu.VMEM((ZB, DH), jnp.float32),
        pltpu.VMEM((ZB, DH), jnp.float32),
        pltpu.VMEM_SHARED((NPAD, DH), jnp.float32),
        pltpu.SemaphoreType.DMA,
        pltpu.SemaphoreType.DMA,
    ],
)
def _sc_rowpass2(ta, tb, src_hbm, dst_hbm, outa, outb, *rest):
    _sc_rowpass2_body(ta, tb, src_hbm, dst_hbm, outa, outb, *rest)


def _safe_rsqrt(d):
    return jnp.where(d > 0, lax.rsqrt(jnp.where(d > 0, d, 1.0)), 0.0)


def _stage_a_body(x_ref, w_ref, wd0_ref, wd1_ref, keep_ref,
                  xs0_ref, xs1_ref, dc_ref, di_ref):
    wd = wd0_ref[...] + wd1_ref[...]
    keep = keep_ref[...]
    dc = _safe_rsqrt(keep * (wd + 1.0))
    di = _safe_rsqrt(keep * wd)
    dc_ref[...] = dc
    di_ref[...] = di
    xs = jnp.dot(x_ref[...], w_ref[...], preferred_element_type=jnp.float32)
    xs = xs * dc
    xs0_ref[...] = xs[:, :DH]
    xs1_ref[...] = xs[:, DH:]


def _stage_a(x, w, wd0, wd1, keep):
    return pl.pallas_call(
        _stage_a_body,
        grid=(NBLK,),
        in_specs=[
            pl.BlockSpec((RB, D), lambda i: (i, 0)),
            pl.BlockSpec((D, D), lambda i: (0, 0)),
            pl.BlockSpec((RB, 1), lambda i: (i, 0)),
            pl.BlockSpec((RB, 1), lambda i: (i, 0)),
            pl.BlockSpec((RB, 1), lambda i: (i, 0)),
        ],
        out_specs=[
            pl.BlockSpec((RB, DH), lambda i: (i, 0)),
            pl.BlockSpec((RB, DH), lambda i: (i, 0)),
            pl.BlockSpec((RB, 1), lambda i: (i, 0)),
            pl.BlockSpec((RB, 1), lambda i: (i, 0)),
        ],
        out_shape=[
            jax.ShapeDtypeStruct((NPAD, DH), jnp.float32),
            jax.ShapeDtypeStruct((NPAD, DH), jnp.float32),
            jax.ShapeDtypeStruct((NPAD, 1), jnp.float32),
            jax.ShapeDtypeStruct((NPAD, 1), jnp.float32),
        ],
    )(x, w, wd0, wd1, keep)


def _stage_b_body(a0_ref, a1_ref, xs0_ref, xs1_ref, dc_ref, di_ref, b_ref,
                  h_ref, hsb0_ref, hsb1_ref):
    acc = jnp.concatenate([a0_ref[0] + a0_ref[1], a1_ref[0] + a1_ref[1]],
                          axis=1)
    xs = jnp.concatenate([xs0_ref[...], xs1_ref[...]], axis=1)
    h = jax.nn.relu(dc_ref[...] * (acc + xs) + b_ref[0:1, :])
    h_ref[...] = h
    hsb = di_ref[...] * h
    hsb0_ref[...] = hsb[:, :DH]
    hsb1_ref[...] = hsb[:, DH:]


def _stage_b(a0, a1, xs0, xs1, dc, di, b8):
    return pl.pallas_call(
        _stage_b_body,
        grid=(NBLK,),
        in_specs=[
            pl.BlockSpec((NC, RB, DH), lambda i: (0, i, 0)),
            pl.BlockSpec((NC, RB, DH), lambda i: (0, i, 0)),
            pl.BlockSpec((RB, DH), lambda i: (i, 0)),
            pl.BlockSpec((RB, DH), lambda i: (i, 0)),
            pl.BlockSpec((RB, 1), lambda i: (i, 0)),
            pl.BlockSpec((RB, 1), lambda i: (i, 0)),
            pl.BlockSpec((8, D), lambda i: (0, 0)),
        ],
        out_specs=[
            pl.BlockSpec((RB, D), lambda i: (i, 0)),
            pl.BlockSpec((RB, DH), lambda i: (i, 0)),
            pl.BlockSpec((RB, DH), lambda i: (i, 0)),
        ],
        out_shape=[
            jax.ShapeDtypeStruct((NPAD, D), jnp.float32),
            jax.ShapeDtypeStruct((NPAD, DH), jnp.float32),
            jax.ShapeDtypeStruct((NPAD, DH), jnp.float32),
        ],
    )(a0, a1, xs0, xs1, dc, di, b8)


def _stage_c_body(b0_ref, b1_ref, h_ref, di_ref, keep_ref, s_ref):
    accb = jnp.concatenate([b0_ref[0] + b0_ref[1], b1_ref[0] + b1_ref[1]],
                           axis=1)
    agg = h_ref[...] - di_ref[...] * accb
    s = jnp.sum(jnp.abs(agg), axis=1, keepdims=True)
    s_ref[...] = jnp.where(keep_ref[...] > 0, s, -jnp.inf)


def _stage_c(b0, b1, h, di, keep):
    return pl.pallas_call(
        _stage_c_body,
        grid=(NBLK,),
        in_specs=[
            pl.BlockSpec((NC, RB, DH), lambda i: (0, i, 0)),
            pl.BlockSpec((NC, RB, DH), lambda i: (0, i, 0)),
            pl.BlockSpec((RB, D), lambda i: (i, 0)),
            pl.BlockSpec((RB, 1), lambda i: (i, 0)),
            pl.BlockSpec((RB, 1), lambda i: (i, 0)),
        ],
        out_specs=pl.BlockSpec((RB, 1), lambda i: (i, 0)),
        out_shape=jax.ShapeDtypeStruct((NPAD, 1), jnp.float32),
    )(b0, b1, h, di, keep)


def _topk_body(s_ref, keep_ref, *, k):
    sb = lax.bitcast_convert_type(s_ref[...], jnp.int32)
    ikey = sb ^ (lax.shift_right_arithmetic(sb, 31) & jnp.int32(0x7FFFFFFF))
    msb = jnp.int32(-2147483648)

    def bit_body(i, t):
        bit = lax.shift_left(jnp.int32(1), 31 - i)
        t2 = t | bit
        thr = t2 ^ msb
        cnt = jnp.sum((ikey >= thr).astype(jnp.int32))
        return jnp.where(cnt >= k, t2, t)

    t = lax.fori_loop(0, 32, bit_body, jnp.int32(0))
    thr = t ^ msb
    gt = ikey > thr
    tie = ikey == thr
    need = (k - jnp.sum(gt.astype(jnp.int32))).astype(jnp.float32)

    tie_f = tie.astype(jnp.float32)
    c_i = lax.broadcasted_iota(jnp.int32, (D, D), 0)
    c_j = lax.broadcasted_iota(jnp.int32, (D, D), 1)
    u_strict = (c_i < c_j).astype(jnp.float32)
    prefix = jnp.dot(tie_f, u_strict, preferred_element_type=jnp.float32)
    r_i = lax.broadcasted_iota(jnp.int32, (NPAD // D, NPAD // D), 0)
    r_j = lax.broadcasted_iota(jnp.int32, (NPAD // D, NPAD // D), 1)
    l_strict = (r_j < r_i).astype(jnp.float32)
    rowtot = jnp.sum(tie_f, axis=1, keepdims=True)
    offs = jnp.dot(l_strict, rowtot, preferred_element_type=jnp.float32)
    rank = offs + prefix
    keep = gt | (tie & (rank < need))
    keep_ref[...] = keep.astype(jnp.float32)


def _topk(score80, k):
    return pl.pallas_call(
        functools.partial(_topk_body, k=k),
        out_shape=jax.ShapeDtypeStruct((NPAD // D, D), jnp.float32),
    )(score80)


def _readout_body(h_ref, keep_ref, o_ref):
    @pl.when(pl.program_id(0) == 0)
    def _():
        o_ref[...] = jnp.zeros((8, D), jnp.float32)
        o_ref[0:1, :] = jnp.full((1, D), -jnp.inf, jnp.float32)

    m = keep_ref[...] > 0
    hm = jnp.where(m, h_ref[...], -jnp.inf)
    hs = jnp.where(m, h_ref[...], 0.0)
    o_ref[0:1, :] = jnp.maximum(o_ref[0:1, :], jnp.max(hm, axis=0, keepdims=True))
    o_ref[1:2, :] = o_ref[1:2, :] + jnp.sum(hs, axis=0, keepdims=True)


def _readout(h, keep):
    return pl.pallas_call(
        _readout_body,
        grid=(NBLK,),
        in_specs=[
            pl.BlockSpec((RB, D), lambda i: (i, 0)),
            pl.BlockSpec((RB, 1), lambda i: (i, 0)),
        ],
        out_specs=pl.BlockSpec((8, D), lambda i: (0, 0)),
        out_shape=jax.ShapeDtypeStruct((8, D), jnp.float32),
    )(h, keep)


def _mlp_body(r1_ref, r2_ref, r3_ref, w1_ref, b1_ref, w2_ref, b2_ref,
              w3_ref, b3_ref, o_ref, *, k1, k2):
    def ro(r_ref, k):
        mx = r_ref[0:1, :]
        mn = r_ref[1:2, :] * (1.0 / k)
        return jnp.concatenate([mx, mn], axis=1)

    x1 = jax.nn.relu(ro(r1_ref, k1))
    x2 = jax.nn.relu(ro(r2_ref, k2))
    x3 = jax.nn.relu(ro(r3_ref, k2))
    z = x1 + x2 + x3
    z = jax.nn.relu(jnp.dot(z, w1_ref[...], preferred_element_type=jnp.float32)
                    + b1_ref[0:1, :])
    z = jax.nn.relu(jnp.dot(z, w2_ref[...], preferred_element_type=jnp.float32)
                    + b2_ref[0:1, :])
    z = jnp.dot(z, w3_ref[...], preferred_element_type=jnp.float32) + b3_ref[0:1, :]
    col = lax.broadcasted_iota(jnp.int32, (1, D), 1)
    zm = jnp.where(col < 6, z, -jnp.inf)
    mx = jnp.max(zm, axis=1, keepdims=True)
    ze = jnp.where(col < 6, jnp.exp(zm - mx), 0.0)
    lse = jnp.log(jnp.sum(ze, axis=1, keepdims=True)) + mx
    o_ref[...] = zm - lse


def _mlp(r1, r2, r3, w1p, b1p, w2p, b2p, w3p, b3p, k1, k2):
    return pl.pallas_call(
        functools.partial(_mlp_body, k1=float(k1), k2=float(k2)),
        out_shape=jax.ShapeDtypeStruct((1, D), jnp.float32),
    )(r1, r2, r3, w1p, b1p, w2p, b2p, w3p, b3p)


def _pad_rows(a, rows):
    return jnp.pad(a, ((0, rows - a.shape[0]), (0, 0)))


def kernel(x, edge_index, batch, W1, b1, W2, b2, W3, b3,
           lin1_W, lin1_b, lin2_W, lin2_b, lin3_W, lin3_b):
    src = edge_index[0].astype(jnp.int32)
    dst = edge_index[1].astype(jnp.int32)
    src_t = src.reshape(NW, NCHUNK, RCH)
    dst_t = dst.reshape(NW, NCHUNK, RCH)

    xp = _pad_rows(x, NPAD)
    ones = jnp.ones((NPAD, 1), jnp.float32)

    k1 = int(math.ceil(0.5 * N))
    k2 = int(math.ceil(0.5 * k1))

    def b_pad8(b):
        return jnp.broadcast_to(b[None, :], (8, b.shape[0]))

    def layer(h_in, W, b, keep_col, want_score):
        keep16 = jnp.broadcast_to(keep_col, (NPAD, DG))
        wd = _sc_rowpass(keep16, dst_t, src_t)
        xs0, xs1, dc, di = _stage_a(h_in, W, wd[0, :, 0:1], wd[1, :, 0:1],
                                    keep_col)
        a0, a1 = _sc_rowpass2(xs0, xs1, src_t, dst_t)
        h, hsb0, hsb1 = _stage_b(a0, a1, xs0, xs1, dc, di, b_pad8(b))
        if not want_score:
            return h, None
        b0, b1 = _sc_rowpass2(hsb0, hsb1, src_t, dst_t)
        score = _stage_c(b0, b1, h, di, keep_col)
        return h, score

    h1, s1 = layer(xp, W1, b1, ones, True)
    keep1 = _topk(s1.reshape(NPAD // D, D), k1).reshape(NPAD, 1)
    h2, s2 = layer(h1, W2, b2, keep1, True)
    keep2 = _topk(s2.reshape(NPAD // D, D), k2).reshape(NPAD, 1)
    h3, _ = layer(h2, W3, b3, keep2, False)

    r1 = _readout(h1, keep1)
    r2 = _readout(h2, keep2)
    r3 = _readout(h3, keep2)

    w3p = jnp.pad(lin3_W, ((0, 0), (0, D - lin3_W.shape[1])))
    b3p = jnp.pad(lin3_b, (0, D - lin3_b.shape[0]))
    w2p = lin2_W
    out = _mlp(r1, r2, r3,
               lin1_W, b_pad8(lin1_b),
               w2p, b_pad8(jnp.pad(lin2_b, (0, 0))),
               w3p, b_pad8(b3p), k1, k2)
    return out[:, :6]


# speedup vs baseline: 1.0368x; 1.0182x over previous
import functools
import math

import jax
import jax.numpy as jnp
from jax import lax
from jax.experimental import pallas as pl
from jax.experimental.pallas import tpu as pltpu
from jax.experimental.pallas import tpu_sc as plsc

N = 10000
E = 320000
D = 128
NPAD = 10240
RB = 2048
NBLK = NPAD // RB
NC = 2
NS = 16
NW = NC * NS


_mesh = plsc.VectorSubcoreMesh(core_axis_name="c", subcore_axis_name="s")

RCH = 500
NCHUNK = E // NW // RCH
RPT = NPAD // NS
ZB = 40


DH = 64
DG = 16


def _sc_rowpass_body(dw, table_hbm, src_hbm, dst_hbm, out_hbm,
                     src_v, dst_v, rows0_v, rows1_v, zb_v, acc_sp,
                     sem0, sem1):
    cid = lax.axis_index("c")
    sid = lax.axis_index("s")
    wid = cid * NS + sid

    def _zero_zb(i, _):
        for l in range(dw // 16):
            zb_v[i, pl.ds(l * 16, 16)] = jnp.zeros((16,), jnp.float32)
        return 0
    lax.fori_loop(0, ZB, _zero_zb, 0)

    row0 = sid * RPT
    for j in range(RPT // ZB):
        pltpu.sync_copy(zb_v, acc_sp.at[pl.ds(row0 + j * ZB, ZB)])

    pltpu.sync_copy(src_hbm.at[wid], src_v)
    pltpu.sync_copy(dst_hbm.at[wid], dst_v)
    plsc.subcore_barrier()

    bufs = (rows0_v, rows1_v)
    sems = (sem0, sem1)
    g = pltpu.async_copy(table_hbm.at[src_v.at[0]], bufs[0], sems[0])
    for j in range(NCHUNK):
        g.wait()
        if j + 1 < NCHUNK:
            g = pltpu.async_copy(table_hbm.at[src_v.at[j + 1]],
                                 bufs[(j + 1) % 2], sems[(j + 1) % 2])
        pltpu.sync_copy(bufs[j % 2], acc_sp.at[dst_v.at[j]], add=True)
    plsc.subcore_barrier()

    pltpu.sync_copy(acc_sp.at[pl.ds(row0, RPT)],
                    out_hbm.at[cid, pl.ds(row0, RPT)])


@functools.cache
def _make_rowpass(dw):
    @functools.partial(
        pl.kernel, mesh=_mesh,
        out_type=jax.ShapeDtypeStruct((NC, NPAD, dw), jnp.float32),
        compiler_params=pltpu.CompilerParams(use_tc_tiling_on_sc=False),
        scratch_types=[
            pltpu.VMEM((NCHUNK, RCH), jnp.int32),
            pltpu.VMEM((NCHUNK, RCH), jnp.int32),
            pltpu.VMEM((RCH, dw), jnp.float32),
            pltpu.VMEM((RCH, dw), jnp.float32),
            pltpu.VMEM((ZB, dw), jnp.float32),
            pltpu.VMEM_SHARED((NPAD, dw), jnp.float32),
            pltpu.SemaphoreType.DMA,
            pltpu.SemaphoreType.DMA,
        ],
    )
    def _k(table_hbm, src_hbm, dst_hbm, out_hbm, *rest):
        _sc_rowpass_body(dw, table_hbm, src_hbm, dst_hbm, out_hbm, *rest)

    return _k


def _sc_rowpass(table, src_t, dst_t):
    return _make_rowpass(table.shape[1])(table, src_t, dst_t)


def _sc_rowpass2_body(ta_hbm, tb_hbm, src_hbm, dst_hbm, outa_hbm, outb_hbm,
                      src_v, dst_v, rows0_v, rows1_v, zb_v, wb_v, acc_sp,
                      sem0, sem1):
    cid = lax.axis_index("c")
    sid = lax.axis_index("s")
    wid = cid * NS + sid
    row0 = sid * RPT

    def _zero_zb(i, _):
        for l in range(DH // 16):
            zb_v[i, pl.ds(l * 16, 16)] = jnp.zeros((16,), jnp.float32)
        return 0
    lax.fori_loop(0, ZB, _zero_zb, 0)

    def _zero_acc():
        for j in range(RPT // ZB):
            pltpu.sync_copy(zb_v, acc_sp.at[pl.ds(row0 + j * ZB, ZB)])

    def _writeback(out_hbm):
        pltpu.sync_copy(acc_sp.at[pl.ds(row0, RPT)],
                        out_hbm.at[cid, pl.ds(row0, RPT)])

    _zero_acc()
    pltpu.sync_copy(src_hbm.at[wid], src_v)
    pltpu.sync_copy(dst_hbm.at[wid], dst_v)
    plsc.subcore_barrier()

    bufs = (rows0_v, rows1_v)
    sems = (sem0, sem1)
    tabs = (ta_hbm, tb_hbm)
    g = pltpu.async_copy(ta_hbm.at[src_v.at[0]], bufs[0], sems[0])
    for j in range(2 * NCHUNK):
        g.wait()
        if j + 1 < 2 * NCHUNK:
            nj = (j + 1) % NCHUNK
            g = pltpu.async_copy(tabs[(j + 1) // NCHUNK].at[src_v.at[nj]],
                                 bufs[(j + 1) % 2], sems[(j + 1) % 2])
        if j == NCHUNK:
            plsc.subcore_barrier()
            _writeback(outa_hbm)
            _zero_acc()
            plsc.subcore_barrier()
        pltpu.sync_copy(bufs[j % 2], acc_sp.at[dst_v.at[j % NCHUNK]],
                        add=True)
    plsc.subcore_barrier()
    _writeback(outb_hbm)


@functools.partial(
    pl.kernel, mesh=_mesh,
    out_type=[jax.ShapeDtypeStruct((NC, NPAD, DH), jnp.float32),
              jax.ShapeDtypeStruct((NC, NPAD, DH), jnp.float32)],
    compiler_params=pltpu.CompilerParams(use_tc_tiling_on_sc=False),
    scratch_types=[
        pltpu.VMEM((NCHUNK, RCH), jnp.int32),
        pltpu.VMEM((NCHUNK, RCH), jnp.int32),
        pltpu.VMEM((RCH, DH), jnp.float32),
        pltpu.VMEM((RCH, DH), jnp.float32),
        pltpu.VMEM((ZB, DH), jnp.float32),
        pltpu.VMEM((ZB, DH), jnp.float32),
        pltpu.VMEM_SHARED((NPAD, DH), jnp.float32),
        pltpu.SemaphoreType.DMA,
        pltpu.SemaphoreType.DMA,
    ],
)
def _sc_rowpass2(ta, tb, src_hbm, dst_hbm, outa, outb, *rest):
    _sc_rowpass2_body(ta, tb, src_hbm, dst_hbm, outa, outb, *rest)


def _safe_rsqrt(d):
    return jnp.where(d > 0, lax.rsqrt(jnp.where(d > 0, d, 1.0)), 0.0)


def _stage_a_body(x_ref, w_ref, wd0_ref, wd1_ref, keep_ref,
                  xs0_ref, xs1_ref, dc_ref, di_ref):
    wd = wd0_ref[...] + wd1_ref[...]
    keep = keep_ref[...]
    dc = _safe_rsqrt(keep * (wd + 1.0))
    di = _safe_rsqrt(keep * wd)
    dc_ref[...] = dc
    di_ref[...] = di
    xs = jnp.dot(x_ref[...], w_ref[...], preferred_element_type=jnp.float32)
    xs = xs * dc
    xs0_ref[...] = xs[:, :DH]
    xs1_ref[...] = xs[:, DH:]


def _stage_a(x, w, wd0, wd1, keep):
    return pl.pallas_call(
        _stage_a_body,
        grid=(NBLK,),
        in_specs=[
            pl.BlockSpec((RB, D), lambda i: (i, 0)),
            pl.BlockSpec((D, D), lambda i: (0, 0)),
            pl.BlockSpec((RB, 1), lambda i: (i, 0)),
            pl.BlockSpec((RB, 1), lambda i: (i, 0)),
            pl.BlockSpec((RB, 1), lambda i: (i, 0)),
        ],
        out_specs=[
            pl.BlockSpec((RB, DH), lambda i: (i, 0)),
            pl.BlockSpec((RB, DH), lambda i: (i, 0)),
            pl.BlockSpec((RB, 1), lambda i: (i, 0)),
            pl.BlockSpec((RB, 1), lambda i: (i, 0)),
        ],
        out_shape=[
            jax.ShapeDtypeStruct((NPAD, DH), jnp.float32),
            jax.ShapeDtypeStruct((NPAD, DH), jnp.float32),
            jax.ShapeDtypeStruct((NPAD, 1), jnp.float32),
            jax.ShapeDtypeStruct((NPAD, 1), jnp.float32),
        ],
    )(x, w, wd0, wd1, keep)


def _stage_b_body(a0_ref, a1_ref, xs0_ref, xs1_ref, dc_ref, di_ref, b_ref,
                  h_ref, hsb0_ref, hsb1_ref):
    acc = jnp.concatenate([a0_ref[0] + a0_ref[1], a1_ref[0] + a1_ref[1]],
                          axis=1)
    xs = jnp.concatenate([xs0_ref[...], xs1_ref[...]], axis=1)
    h = jax.nn.relu(dc_ref[...] * (acc + xs) + b_ref[0:1, :])
    h_ref[...] = h
    hsb = di_ref[...] * h
    hsb0_ref[...] = hsb[:, :DH]
    hsb1_ref[...] = hsb[:, DH:]


def _stage_b(a0, a1, xs0, xs1, dc, di, b8):
    return pl.pallas_call(
        _stage_b_body,
        grid=(NBLK,),
        in_specs=[
            pl.BlockSpec((NC, RB, DH), lambda i: (0, i, 0)),
            pl.BlockSpec((NC, RB, DH), lambda i: (0, i, 0)),
            pl.BlockSpec((RB, DH), lambda i: (i, 0)),
            pl.BlockSpec((RB, DH), lambda i: (i, 0)),
            pl.BlockSpec((RB, 1), lambda i: (i, 0)),
            pl.BlockSpec((RB, 1), lambda i: (i, 0)),
            pl.BlockSpec((8, D), lambda i: (0, 0)),
        ],
        out_specs=[
            pl.BlockSpec((RB, D), lambda i: (i, 0)),
            pl.BlockSpec((RB, DH), lambda i: (i, 0)),
            pl.BlockSpec((RB, DH), lambda i: (i, 0)),
        ],
        out_shape=[
            jax.ShapeDtypeStruct((NPAD, D), jnp.float32),
            jax.ShapeDtypeStruct((NPAD, DH), jnp.float32),
            jax.ShapeDtypeStruct((NPAD, DH), jnp.float32),
        ],
    )(a0, a1, xs0, xs1, dc, di, b8)


def _stage_c_body(b0_ref, b1_ref, h_ref, di_ref, keep_ref, s_ref):
    accb = jnp.concatenate([b0_ref[0] + b0_ref[1], b1_ref[0] + b1_ref[1]],
                           axis=1)
    agg = h_ref[...] - di_ref[...] * accb
    s = jnp.sum(jnp.abs(agg), axis=1, keepdims=True)
    s_ref[...] = jnp.where(keep_ref[...] > 0, s, -jnp.inf)


def _stage_c(b0, b1, h, di, keep):
    return pl.pallas_call(
        _stage_c_body,
        grid=(NBLK,),
        in_specs=[
            pl.BlockSpec((NC, RB, DH), lambda i: (0, i, 0)),
            pl.BlockSpec((NC, RB, DH), lambda i: (0, i, 0)),
            pl.BlockSpec((RB, D), lambda i: (i, 0)),
            pl.BlockSpec((RB, 1), lambda i: (i, 0)),
            pl.BlockSpec((RB, 1), lambda i: (i, 0)),
        ],
        out_specs=pl.BlockSpec((RB, 1), lambda i: (i, 0)),
        out_shape=jax.ShapeDtypeStruct((NPAD, 1), jnp.float32),
    )(b0, b1, h, di, keep)


def _topk_body(s_ref, keep_ref, *, k):
    sb = lax.bitcast_convert_type(s_ref[...], jnp.int32)
    ikey = sb ^ (lax.shift_right_arithmetic(sb, 31) & jnp.int32(0x7FFFFFFF))
    msb = jnp.int32(-2147483648)

    def bit_body(i, t):
        bit = lax.shift_left(jnp.int32(1), 31 - i)
        t2 = t | bit
        thr = t2 ^ msb
        cnt = jnp.sum((ikey >= thr).astype(jnp.int32))
        return jnp.where(cnt >= k, t2, t)

    t = lax.fori_loop(0, 32, bit_body, jnp.int32(0))
    thr = t ^ msb
    gt = ikey > thr
    tie = ikey == thr
    need = (k - jnp.sum(gt.astype(jnp.int32))).astype(jnp.float32)

    tie_f = tie.astype(jnp.float32)
    c_i = lax.broadcasted_iota(jnp.int32, (D, D), 0)
    c_j = lax.broadcasted_iota(jnp.int32, (D, D), 1)
    u_strict = (c_i < c_j).astype(jnp.float32)
    prefix = jnp.dot(tie_f, u_strict, preferred_element_type=jnp.float32)
    r_i = lax.broadcasted_iota(jnp.int32, (NPAD // D, NPAD // D), 0)
    r_j = lax.broadcasted_iota(jnp.int32, (NPAD // D, NPAD // D), 1)
    l_strict = (r_j < r_i).astype(jnp.float32)
    rowtot = jnp.sum(tie_f, axis=1, keepdims=True)
    offs = jnp.dot(l_strict, rowtot, preferred_element_type=jnp.float32)
    rank = offs + prefix
    keep = gt | (tie & (rank < need))
    keep_ref[...] = keep.astype(jnp.float32)


def _topk(score80, k):
    return pl.pallas_call(
        functools.partial(_topk_body, k=k),
        out_shape=jax.ShapeDtypeStruct((NPAD // D, D), jnp.float32),
    )(score80)


def _stage_b_ro_body(a0_ref, a1_ref, xs0_ref, xs1_ref, dc_ref, keep_ref,
                     b_ref, o_ref):
    @pl.when(pl.program_id(0) == 0)
    def _():
        o_ref[...] = jnp.zeros((8, D), jnp.float32)
        o_ref[0:1, :] = jnp.full((1, D), -jnp.inf, jnp.float32)

    acc = jnp.concatenate([a0_ref[0] + a0_ref[1], a1_ref[0] + a1_ref[1]],
                          axis=1)
    xs = jnp.concatenate([xs0_ref[...], xs1_ref[...]], axis=1)
    h = jax.nn.relu(dc_ref[...] * (acc + xs) + b_ref[0:1, :])
    m = keep_ref[...] > 0
    hm = jnp.where(m, h, -jnp.inf)
    hs = jnp.where(m, h, 0.0)
    o_ref[0:1, :] = jnp.maximum(o_ref[0:1, :], jnp.max(hm, axis=0, keepdims=True))
    o_ref[1:2, :] = o_ref[1:2, :] + jnp.sum(hs, axis=0, keepdims=True)


def _stage_b_ro(a0, a1, xs0, xs1, dc, keep, b8):
    return pl.pallas_call(
        _stage_b_ro_body,
        grid=(NBLK,),
        in_specs=[
            pl.BlockSpec((NC, RB, DH), lambda i: (0, i, 0)),
            pl.BlockSpec((NC, RB, DH), lambda i: (0, i, 0)),
            pl.BlockSpec((RB, DH), lambda i: (i, 0)),
            pl.BlockSpec((RB, DH), lambda i: (i, 0)),
            pl.BlockSpec((RB, 1), lambda i: (i, 0)),
            pl.BlockSpec((RB, 1), lambda i: (i, 0)),
            pl.BlockSpec((8, D), lambda i: (0, 0)),
        ],
        out_specs=pl.BlockSpec((8, D), lambda i: (0, 0)),
        out_shape=jax.ShapeDtypeStruct((8, D), jnp.float32),
    )(a0, a1, xs0, xs1, dc, keep, b8)


def _readout_body(h_ref, keep_ref, o_ref):
    @pl.when(pl.program_id(0) == 0)
    def _():
        o_ref[...] = jnp.zeros((8, D), jnp.float32)
        o_ref[0:1, :] = jnp.full((1, D), -jnp.inf, jnp.float32)

    m = keep_ref[...] > 0
    hm = jnp.where(m, h_ref[...], -jnp.inf)
    hs = jnp.where(m, h_ref[...], 0.0)
    o_ref[0:1, :] = jnp.maximum(o_ref[0:1, :], jnp.max(hm, axis=0, keepdims=True))
    o_ref[1:2, :] = o_ref[1:2, :] + jnp.sum(hs, axis=0, keepdims=True)


def _readout(h, keep):
    return pl.pallas_call(
        _readout_body,
        grid=(NBLK,),
        in_specs=[
            pl.BlockSpec((RB, D), lambda i: (i, 0)),
            pl.BlockSpec((RB, 1), lambda i: (i, 0)),
        ],
        out_specs=pl.BlockSpec((8, D), lambda i: (0, 0)),
        out_shape=jax.ShapeDtypeStruct((8, D), jnp.float32),
    )(h, keep)


def _mlp_body(r1_ref, r2_ref, r3_ref, w1_ref, b1_ref, w2_ref, b2_ref,
              w3_ref, b3_ref, o_ref, *, k1, k2):
    def ro(r_ref, k):
        mx = r_ref[0:1, :]
        mn = r_ref[1:2, :] * (1.0 / k)
        return jnp.concatenate([mx, mn], axis=1)

    x1 = jax.nn.relu(ro(r1_ref, k1))
    x2 = jax.nn.relu(ro(r2_ref, k2))
    x3 = jax.nn.relu(ro(r3_ref, k2))
    z = x1 + x2 + x3
    z = jax.nn.relu(jnp.dot(z, w1_ref[...], preferred_element_type=jnp.float32)
                    + b1_ref[0:1, :])
    z = jax.nn.relu(jnp.dot(z, w2_ref[...], preferred_element_type=jnp.float32)
                    + b2_ref[0:1, :])
    z = jnp.dot(z, w3_ref[...], preferred_element_type=jnp.float32) + b3_ref[0:1, :]
    col = lax.broadcasted_iota(jnp.int32, (1, D), 1)
    zm = jnp.where(col < 6, z, -jnp.inf)
    mx = jnp.max(zm, axis=1, keepdims=True)
    ze = jnp.where(col < 6, jnp.exp(zm - mx), 0.0)
    lse = jnp.log(jnp.sum(ze, axis=1, keepdims=True)) + mx
    o_ref[...] = zm - lse


def _mlp(r1, r2, r3, w1p, b1p, w2p, b2p, w3p, b3p, k1, k2):
    return pl.pallas_call(
        functools.partial(_mlp_body, k1=float(k1), k2=float(k2)),
        out_shape=jax.ShapeDtypeStruct((1, D), jnp.float32),
    )(r1, r2, r3, w1p, b1p, w2p, b2p, w3p, b3p)


def _pad_rows(a, rows):
    return jnp.pad(a, ((0, rows - a.shape[0]), (0, 0)))


def kernel(x, edge_index, batch, W1, b1, W2, b2, W3, b3,
           lin1_W, lin1_b, lin2_W, lin2_b, lin3_W, lin3_b):
    src = edge_index[0].astype(jnp.int32)
    dst = edge_index[1].astype(jnp.int32)
    src_t = src.reshape(NW, NCHUNK, RCH)
    dst_t = dst.reshape(NW, NCHUNK, RCH)

    xp = _pad_rows(x, NPAD)
    ones = jnp.ones((NPAD, 1), jnp.float32)

    k1 = int(math.ceil(0.5 * N))
    k2 = int(math.ceil(0.5 * k1))

    def b_pad8(b):
        return jnp.broadcast_to(b[None, :], (8, b.shape[0]))

    def layer(h_in, W, b, keep_col, want_score):
        keep16 = jnp.broadcast_to(keep_col, (NPAD, DG))
        wd = _sc_rowpass(keep16, dst_t, src_t)
        xs0, xs1, dc, di = _stage_a(h_in, W, wd[0, :, 0:1], wd[1, :, 0:1],
                                    keep_col)
        a0, a1 = _sc_rowpass2(xs0, xs1, src_t, dst_t)
        if not want_score:
            return _stage_b_ro(a0, a1, xs0, xs1, dc, keep_col, b_pad8(b))
        h, hsb0, hsb1 = _stage_b(a0, a1, xs0, xs1, dc, di, b_pad8(b))
        b0, b1 = _sc_rowpass2(hsb0, hsb1, src_t, dst_t)
        score = _stage_c(b0, b1, h, di, keep_col)
        return h, score

    h1, s1 = layer(xp, W1, b1, ones, True)
    keep1 = _topk(s1.reshape(NPAD // D, D), k1).reshape(NPAD, 1)
    h2, s2 = layer(h1, W2, b2, keep1, True)
    keep2 = _topk(s2.reshape(NPAD // D, D), k2).reshape(NPAD, 1)
    r3 = layer(h2, W3, b3, keep2, False)

    r1 = _readout(h1, keep1)
    r2 = _readout(h2, keep2)

    w3p = jnp.pad(lin3_W, ((0, 0), (0, D - lin3_W.shape[1])))
    b3p = jnp.pad(lin3_b, (0, D - lin3_b.shape[0]))
    w2p = lin2_W
    out = _mlp(r1, r2, r3,
               lin1_W, b_pad8(lin1_b),
               w2p, b_pad8(jnp.pad(lin2_b, (0, 0))),
               w3p, b_pad8(b3p), k1, k2)
    return out[:, :6]
